# Initial kernel scaffold; baseline (speedup 1.0000x reference)
import jax
import jax.numpy as jnp
from jax.experimental import pallas as pl


def kernel(z, W0, b0, W1, b1, W2, b2, W3, b3, Wout, bout, codebook):
    raise NotImplementedError("write your pallas kernel here")


# speedup vs baseline: 1.0284x; 1.0284x over previous
import functools

import jax
import jax.numpy as jnp
from jax import lax
from jax.experimental import pallas as pl
from jax.experimental.pallas import tpu as pltpu
from jax.experimental.pallas import tpu_sc as plsc

B, HW = 8, 576
INPUT_DIM, HIDDEN, LATENT, NUM_EMBED = 64, 512, 256, 8192
OUT_DIM = LATENT + 2
BETA = 0.25
ROWS = B * HW

M_BLK = 512
M_GRID = ROWS // M_BLK
N_CHUNK = 1024
N_CHUNKS = NUM_EMBED // N_CHUNK

NW = 32
ROWS_PER_W = ROWS // NW
GCHUNK = 72


def _decode_argmin_body(z_ref, w0, b0, w1, b1, w2, b2, w3, b3,
                        wl, bl, ws, bs_, cb_ref,
                        zl_ref, sr_ref, ids_ref):
    x = z_ref[...]
    x = jnp.maximum(jnp.dot(x, w0[...], preferred_element_type=jnp.float32) + b0[...], 0.0)
    x = jnp.maximum(jnp.dot(x, w1[...], preferred_element_type=jnp.float32) + b1[...], 0.0)
    x = jnp.maximum(jnp.dot(x, w2[...], preferred_element_type=jnp.float32) + b2[...], 0.0)
    x = jnp.maximum(jnp.dot(x, w3[...], preferred_element_type=jnp.float32) + b3[...], 0.0)
    zl = jnp.dot(x, wl[...], preferred_element_type=jnp.float32) + bl[...]
    sr = jnp.dot(x, ws[...], preferred_element_type=jnp.float32) + bs_[...]
    zl_ref[...] = zl
    sr_ref[...] = sr

    s1 = jnp.sum(zl ** 2, axis=1, keepdims=True)
    run_min = jnp.full((M_BLK,), jnp.inf, dtype=jnp.float32)
    run_idx = jnp.zeros((M_BLK,), dtype=jnp.int32)
    for c in range(N_CHUNKS):
        cb_c = cb_ref[:, pl.ds(c * N_CHUNK, N_CHUNK)]
        s2 = jnp.sum(cb_c ** 2, axis=0)[None, :]
        m = jnp.dot(zl, cb_c, preferred_element_type=jnp.float32)
        d = s1 + s2 - 2.0 * m
        cmin = jnp.min(d, axis=1)
        col = lax.broadcasted_iota(jnp.int32, (M_BLK, N_CHUNK), 1) + (c * N_CHUNK)
        cand = jnp.where(d == cmin[:, None], col, jnp.int32(2 ** 30))
        cidx = jnp.min(cand, axis=1)
        better = cmin < run_min
        run_idx = jnp.where(better, cidx, run_idx)
        run_min = jnp.where(better, cmin, run_min)
    ids_ref[...] = run_idx[None, None, :]


def _transpose_body(cb_ref, out_ref):
    out_ref[...] = cb_ref[...].T


def _sc_gather_body(cbt_hbm, ids_hbm, zl_hbm, out_hbm, part_hbm,
                    idx_v, zq_v, zl_v, acc_v, sem):
    wid = lax.axis_index("s") * 2 + lax.axis_index("c")
    base = wid * ROWS_PER_W
    pltpu.sync_copy(ids_hbm.at[pl.ds(base, ROWS_PER_W)], idx_v)
    pltpu.sync_copy(zl_hbm.at[pl.ds(base, ROWS_PER_W)], zl_v)
    for g in range(ROWS_PER_W // GCHUNK):
        pltpu.async_copy(
            cbt_hbm.at[idx_v.at[pl.ds(g * GCHUNK, GCHUNK)]],
            zq_v.at[pl.ds(g * GCHUNK, GCHUNK)], sem).wait()

    def row_step(r, acc):
        for c in range(LATENT // 16):
            sl = pl.ds(c * 16, 16)
            zq = zq_v[r, sl]
            zlv = zl_v[r, sl]
            t = zq - zlv
            zq_v[r, sl] = zlv + t
            acc = acc + t * t
        return acc

    acc = lax.fori_loop(0, ROWS_PER_W, row_step,
                        jnp.zeros((16,), dtype=jnp.float32))
    acc_v[...] = acc
    pltpu.sync_copy(zq_v, out_hbm.at[pl.ds(base, ROWS_PER_W)])
    pltpu.sync_copy(acc_v, part_hbm.at[wid])


def _sc_gather(cbt, ids, zl):
    run = functools.partial(
        pl.kernel,
        out_type=(
            jax.ShapeDtypeStruct((ROWS, LATENT), jnp.float32),
            jax.ShapeDtypeStruct((NW, 16), jnp.float32),
        ),
        mesh=plsc.VectorSubcoreMesh(core_axis_name="c", subcore_axis_name="s",
                                    num_cores=2),
        scratch_types=[
            pltpu.VMEM((ROWS_PER_W,), jnp.int32),
            pltpu.VMEM((ROWS_PER_W, LATENT), jnp.float32),
            pltpu.VMEM((ROWS_PER_W, LATENT), jnp.float32),
            pltpu.VMEM((16,), jnp.float32),
            pltpu.SemaphoreType.DMA,
        ],
    )(_sc_gather_body)
    return run(cbt, ids, zl)


@jax.jit
def kernel(z, W0, b0, W1, b1, W2, b2, W3, b3, Wout, bout, codebook):
    zf = z.reshape(ROWS, INPUT_DIM)
    wl, ws = Wout[:, :LATENT], Wout[:, LATENT:]
    bl, bs_ = bout[:LATENT][None, :], bout[LATENT:][None, :]

    full = lambda shape: pl.BlockSpec(shape, lambda i: (0,) * len(shape))
    zl_out, sr_out, ids_out = pl.pallas_call(
        _decode_argmin_body,
        grid=(M_GRID,),
        in_specs=[
            pl.BlockSpec((M_BLK, INPUT_DIM), lambda i: (i, 0)),
            full((INPUT_DIM, HIDDEN)), full((1, HIDDEN)),
            full((HIDDEN, HIDDEN)), full((1, HIDDEN)),
            full((HIDDEN, HIDDEN)), full((1, HIDDEN)),
            full((HIDDEN, HIDDEN)), full((1, HIDDEN)),
            full((HIDDEN, LATENT)), full((1, LATENT)),
            full((HIDDEN, 2)), full((1, 2)),
            full((LATENT, NUM_EMBED)),
        ],
        out_specs=[
            pl.BlockSpec((M_BLK, LATENT), lambda i: (i, 0)),
            pl.BlockSpec((M_BLK, 2), lambda i: (i, 0)),
            pl.BlockSpec((1, 1, M_BLK), lambda i: (i, 0, 0)),
        ],
        out_shape=[
            jax.ShapeDtypeStruct((ROWS, LATENT), jnp.float32),
            jax.ShapeDtypeStruct((ROWS, 2), jnp.float32),
            jax.ShapeDtypeStruct((M_GRID, 1, M_BLK), jnp.int32),
        ],
    )(zf, W0, b0[None, :], W1, b1[None, :], W2, b2[None, :], W3, b3[None, :],
      wl, bl, ws, bs_, codebook)

    cbt = pl.pallas_call(
        _transpose_body,
        grid=(NUM_EMBED // 512,),
        in_specs=[pl.BlockSpec((LATENT, 512), lambda i: (0, i))],
        out_specs=pl.BlockSpec((512, LATENT), lambda i: (i, 0)),
        out_shape=jax.ShapeDtypeStruct((NUM_EMBED, LATENT), jnp.float32),
    )(codebook)

    ids = ids_out.reshape(ROWS)
    zq_st, part = _sc_gather(cbt, ids, zl_out)

    msq = jnp.sum(part) / jnp.float32(ROWS * LATENT)
    loss = msq + msq * BETA

    sr3 = sr_out.reshape(B, HW, 2)
    scaler = sr3[:, 0, 0]
    redshift = sr3[:, 0, 1]
    return (zq_st.reshape(B, HW, LATENT), scaler, redshift, loss, ids)


# speedup vs baseline: 1.2303x; 1.1963x over previous
import functools

import jax
import jax.numpy as jnp
from jax import lax
from jax.experimental import pallas as pl
from jax.experimental.pallas import tpu as pltpu
from jax.experimental.pallas import tpu_sc as plsc

B, HW = 8, 576
INPUT_DIM, HIDDEN, LATENT, NUM_EMBED = 64, 512, 256, 8192
OUT_DIM = LATENT + 2
BETA = 0.25
ROWS = B * HW

M_BLK = 512
M_GRID = ROWS // M_BLK
N_CHUNK = 1024
N_CHUNKS = NUM_EMBED // N_CHUNK

NW = 32
ROWS_PER_W = ROWS // NW
GCHUNK = 72


def _decode_argmin_body(z_ref, w0, b0, w1, b1, w2, b2, w3, b3,
                        wl, bl, ws, bs_, cb_ref,
                        zl_ref, sr_ref, ids_ref, cbt_ref):
    tc = jnp.minimum(pl.program_id(0), N_CHUNKS - 1)
    cbt_ref[...] = cb_ref[:, pl.ds(tc * N_CHUNK, N_CHUNK)].T

    x = z_ref[...]
    x = jnp.maximum(jnp.dot(x, w0[...], preferred_element_type=jnp.float32) + b0[...], 0.0)
    x = jnp.maximum(jnp.dot(x, w1[...], preferred_element_type=jnp.float32) + b1[...], 0.0)
    x = jnp.maximum(jnp.dot(x, w2[...], preferred_element_type=jnp.float32) + b2[...], 0.0)
    x = jnp.maximum(jnp.dot(x, w3[...], preferred_element_type=jnp.float32) + b3[...], 0.0)
    zl = jnp.dot(x, wl[...], preferred_element_type=jnp.float32) + bl[...]
    sr = jnp.dot(x, ws[...], preferred_element_type=jnp.float32) + bs_[...]
    zl_ref[...] = zl
    sr_ref[...] = sr

    s1 = jnp.sum(zl ** 2, axis=1, keepdims=True)
    vmin = jnp.full((M_BLK, N_CHUNK), jnp.inf, dtype=jnp.float32)
    cidx = jnp.zeros((M_BLK, N_CHUNK), dtype=jnp.int32)
    for c in range(N_CHUNKS):
        cb_c = cb_ref[:, pl.ds(c * N_CHUNK, N_CHUNK)]
        s2 = jnp.sum(cb_c ** 2, axis=0)[None, :]
        m = jnp.dot(zl, cb_c, preferred_element_type=jnp.float32)
        d = s1 + s2 - 2.0 * m
        lt = d < vmin
        vmin = jnp.where(lt, d, vmin)
        cidx = jnp.where(lt, c, cidx)
    rowmin = jnp.min(vmin, axis=1)
    col = cidx * N_CHUNK + lax.broadcasted_iota(jnp.int32, (M_BLK, N_CHUNK), 1)
    cand = jnp.where(vmin == rowmin[:, None], col, jnp.int32(2 ** 30))
    ids_ref[...] = jnp.min(cand, axis=1)[None, None, :]


def _sc_gather_body(cbt_hbm, ids_hbm, zl_hbm, out_hbm, part_hbm,
                    idx_v, zq_v, zl_v, acc_v, sem):
    wid = lax.axis_index("s") * 2 + lax.axis_index("c")
    base = wid * ROWS_PER_W
    pltpu.sync_copy(ids_hbm.at[pl.ds(base, ROWS_PER_W)], idx_v)
    pltpu.sync_copy(zl_hbm.at[pl.ds(base, ROWS_PER_W)], zl_v)
    copies = [
        pltpu.async_copy(
            cbt_hbm.at[idx_v.at[pl.ds(g * GCHUNK, GCHUNK)]],
            zq_v.at[pl.ds(g * GCHUNK, GCHUNK)], sem)
        for g in range(ROWS_PER_W // GCHUNK)
    ]
    for cp in copies:
        cp.wait()

    def row_step(r, acc):
        for c in range(LATENT // 16):
            sl = pl.ds(c * 16, 16)
            zq = zq_v[r, sl]
            zlv = zl_v[r, sl]
            t = zq - zlv
            zq_v[r, sl] = zlv + t
            acc = acc + t * t
        return acc

    acc = lax.fori_loop(0, ROWS_PER_W, row_step,
                        jnp.zeros((16,), dtype=jnp.float32))
    acc_v[...] = acc
    pltpu.sync_copy(zq_v, out_hbm.at[pl.ds(base, ROWS_PER_W)])
    pltpu.sync_copy(acc_v, part_hbm.at[wid])


def _sc_gather(cbt, ids, zl):
    run = functools.partial(
        pl.kernel,
        out_type=(
            jax.ShapeDtypeStruct((ROWS, LATENT), jnp.float32),
            jax.ShapeDtypeStruct((NW, 16), jnp.float32),
        ),
        mesh=plsc.VectorSubcoreMesh(core_axis_name="c", subcore_axis_name="s",
                                    num_cores=2),
        scratch_types=[
            pltpu.VMEM((ROWS_PER_W,), jnp.int32),
            pltpu.VMEM((ROWS_PER_W, LATENT), jnp.float32),
            pltpu.VMEM((ROWS_PER_W, LATENT), jnp.float32),
            pltpu.VMEM((16,), jnp.float32),
            pltpu.SemaphoreType.DMA,
        ],
    )(_sc_gather_body)
    return run(cbt, ids, zl)


@jax.jit
def kernel(z, W0, b0, W1, b1, W2, b2, W3, b3, Wout, bout, codebook):
    zf = z.reshape(ROWS, INPUT_DIM)
    wl, ws = Wout[:, :LATENT], Wout[:, LATENT:]
    bl, bs_ = bout[:LATENT][None, :], bout[LATENT:][None, :]

    full = lambda shape: pl.BlockSpec(shape, lambda i: (0,) * len(shape))
    zl_out, sr_out, ids_out, cbt = pl.pallas_call(
        _decode_argmin_body,
        grid=(M_GRID,),
        in_specs=[
            pl.BlockSpec((M_BLK, INPUT_DIM), lambda i: (i, 0)),
            full((INPUT_DIM, HIDDEN)), full((1, HIDDEN)),
            full((HIDDEN, HIDDEN)), full((1, HIDDEN)),
            full((HIDDEN, HIDDEN)), full((1, HIDDEN)),
            full((HIDDEN, HIDDEN)), full((1, HIDDEN)),
            full((HIDDEN, LATENT)), full((1, LATENT)),
            full((HIDDEN, 2)), full((1, 2)),
            full((LATENT, NUM_EMBED)),
        ],
        out_specs=[
            pl.BlockSpec((M_BLK, LATENT), lambda i: (i, 0)),
            pl.BlockSpec((M_BLK, 2), lambda i: (i, 0)),
            pl.BlockSpec((1, 1, M_BLK), lambda i: (i, 0, 0)),
            pl.BlockSpec((N_CHUNK, LATENT),
                         lambda i: (jnp.minimum(i, N_CHUNKS - 1), 0)),
        ],
        out_shape=[
            jax.ShapeDtypeStruct((ROWS, LATENT), jnp.float32),
            jax.ShapeDtypeStruct((ROWS, 2), jnp.float32),
            jax.ShapeDtypeStruct((M_GRID, 1, M_BLK), jnp.int32),
            jax.ShapeDtypeStruct((NUM_EMBED, LATENT), jnp.float32),
        ],
    )(zf, W0, b0[None, :], W1, b1[None, :], W2, b2[None, :], W3, b3[None, :],
      wl, bl, ws, bs_, codebook)

    ids = ids_out.reshape(ROWS)
    zq_st, part = _sc_gather(cbt, ids, zl_out)

    msq = jnp.sum(part) / jnp.float32(ROWS * LATENT)
    loss = msq + msq * BETA

    sr3 = sr_out.reshape(B, HW, 2)
    scaler = sr3[:, 0, 0]
    redshift = sr3[:, 0, 1]
    return (zq_st.reshape(B, HW, LATENT), scaler, redshift, loss, ids)


# speedup vs baseline: 1.3070x; 1.0623x over previous
import functools

import jax
import jax.numpy as jnp
from jax import lax
from jax.experimental import pallas as pl
from jax.experimental.pallas import tpu as pltpu
from jax.experimental.pallas import tpu_sc as plsc

B, HW = 8, 576
INPUT_DIM, HIDDEN, LATENT, NUM_EMBED = 64, 512, 256, 8192
OUT_DIM = LATENT + 2
BETA = 0.25
ROWS = B * HW

M_BLK = 512
M_GRID = ROWS // M_BLK
N_CHUNK = 1024
N_CHUNKS = NUM_EMBED // N_CHUNK

NW = 32
ROWS_PER_W = ROWS // NW
GCHUNK = 72


def _decode_argmin_body(z_ref, w0, b0, w1, b1, w2, b2, w3, b3,
                        wl, bl, ws, bs_, cb_ref,
                        sr_ref, ids_ref, dmin_ref, cbt_ref, s2_ref):
    @pl.when(pl.program_id(0) == 0)
    def _():
        s2_ref[...] = jnp.sum(cb_ref[...] ** 2, axis=0, keepdims=True)

    tc = jnp.minimum(pl.program_id(0), N_CHUNKS - 1)
    cbt_ref[...] = cb_ref[:, pl.ds(tc * N_CHUNK, N_CHUNK)].T

    x = z_ref[...]
    x = jnp.maximum(jnp.dot(x, w0[...], preferred_element_type=jnp.float32) + b0[...], 0.0)
    x = jnp.maximum(jnp.dot(x, w1[...], preferred_element_type=jnp.float32) + b1[...], 0.0)
    x = jnp.maximum(jnp.dot(x, w2[...], preferred_element_type=jnp.float32) + b2[...], 0.0)
    x = jnp.maximum(jnp.dot(x, w3[...], preferred_element_type=jnp.float32) + b3[...], 0.0)
    zl = jnp.dot(x, wl[...], preferred_element_type=jnp.float32) + bl[...]
    sr = jnp.dot(x, ws[...], preferred_element_type=jnp.float32) + bs_[...]
    sr_ref[...] = sr

    s1 = jnp.sum(zl ** 2, axis=1, keepdims=True)
    vmin = jnp.full((M_BLK, N_CHUNK), jnp.inf, dtype=jnp.float32)
    cidx = jnp.zeros((M_BLK, N_CHUNK), dtype=jnp.int32)
    for c in range(N_CHUNKS):
        cb_c = cb_ref[:, pl.ds(c * N_CHUNK, N_CHUNK)]
        s2 = s2_ref[:, pl.ds(c * N_CHUNK, N_CHUNK)]
        m = jnp.dot(zl, cb_c, preferred_element_type=jnp.float32)
        d = s1 + s2 - 2.0 * m
        lt = d < vmin
        vmin = jnp.where(lt, d, vmin)
        cidx = jnp.where(lt, c, cidx)
    rowmin = jnp.min(vmin, axis=1)
    col = cidx * N_CHUNK + lax.broadcasted_iota(jnp.int32, (M_BLK, N_CHUNK), 1)
    cand = jnp.where(vmin == rowmin[:, None], col, jnp.int32(2 ** 30))
    ids_ref[...] = jnp.min(cand, axis=1)[None, None, :]
    dmin_ref[...] = rowmin[None, None, :]


def _sc_gather_body(cbt_hbm, ids_hbm, out_hbm, idx_v, zq_v, sem):
    wid = lax.axis_index("s") * 2 + lax.axis_index("c")
    base = wid * ROWS_PER_W
    pltpu.sync_copy(ids_hbm.at[pl.ds(base, ROWS_PER_W)], idx_v)
    copies = [
        pltpu.async_copy(
            cbt_hbm.at[idx_v.at[pl.ds(g * GCHUNK, GCHUNK)]],
            zq_v.at[pl.ds(g * GCHUNK, GCHUNK)], sem)
        for g in range(ROWS_PER_W // GCHUNK)
    ]
    for cp in copies:
        cp.wait()
    pltpu.sync_copy(zq_v, out_hbm.at[pl.ds(base, ROWS_PER_W)])


def _sc_gather(cbt, ids):
    run = functools.partial(
        pl.kernel,
        out_type=jax.ShapeDtypeStruct((ROWS, LATENT), jnp.float32),
        mesh=plsc.VectorSubcoreMesh(core_axis_name="c", subcore_axis_name="s",
                                    num_cores=2),
        scratch_types=[
            pltpu.VMEM((ROWS_PER_W,), jnp.int32),
            pltpu.VMEM((ROWS_PER_W, LATENT), jnp.float32),
            pltpu.SemaphoreType.DMA,
        ],
    )(_sc_gather_body)
    return run(cbt, ids)


@jax.jit
def kernel(z, W0, b0, W1, b1, W2, b2, W3, b3, Wout, bout, codebook):
    zf = z.reshape(ROWS, INPUT_DIM)
    wl, ws = Wout[:, :LATENT], Wout[:, LATENT:]
    bl, bs_ = bout[:LATENT][None, :], bout[LATENT:][None, :]

    full = lambda shape: pl.BlockSpec(shape, lambda i: (0,) * len(shape))
    sr_out, ids_out, dmin_out, cbt = pl.pallas_call(
        _decode_argmin_body,
        grid=(M_GRID,),
        in_specs=[
            pl.BlockSpec((M_BLK, INPUT_DIM), lambda i: (i, 0)),
            full((INPUT_DIM, HIDDEN)), full((1, HIDDEN)),
            full((HIDDEN, HIDDEN)), full((1, HIDDEN)),
            full((HIDDEN, HIDDEN)), full((1, HIDDEN)),
            full((HIDDEN, HIDDEN)), full((1, HIDDEN)),
            full((HIDDEN, LATENT)), full((1, LATENT)),
            full((HIDDEN, 2)), full((1, 2)),
            full((LATENT, NUM_EMBED)),
        ],
        out_specs=[
            pl.BlockSpec((M_BLK, 2), lambda i: (i, 0)),
            pl.BlockSpec((1, 1, M_BLK), lambda i: (i, 0, 0)),
            pl.BlockSpec((1, 1, M_BLK), lambda i: (i, 0, 0)),
            pl.BlockSpec((N_CHUNK, LATENT),
                         lambda i: (jnp.minimum(i, N_CHUNKS - 1), 0)),
        ],
        out_shape=[
            jax.ShapeDtypeStruct((ROWS, 2), jnp.float32),
            jax.ShapeDtypeStruct((M_GRID, 1, M_BLK), jnp.int32),
            jax.ShapeDtypeStruct((M_GRID, 1, M_BLK), jnp.float32),
            jax.ShapeDtypeStruct((NUM_EMBED, LATENT), jnp.float32),
        ],
        scratch_shapes=[pltpu.VMEM((1, NUM_EMBED), jnp.float32)],
    )(zf, W0, b0[None, :], W1, b1[None, :], W2, b2[None, :], W3, b3[None, :],
      wl, bl, ws, bs_, codebook)

    ids = ids_out.reshape(ROWS)
    zq_st = _sc_gather(cbt, ids)

    msq = jnp.sum(dmin_out) / jnp.float32(ROWS * LATENT)
    loss = msq + msq * BETA

    sr3 = sr_out.reshape(B, HW, 2)
    scaler = sr3[:, 0, 0]
    redshift = sr3[:, 0, 1]
    return (zq_st.reshape(B, HW, LATENT), scaler, redshift, loss, ids)
